# baseline (speedup 1.0000x reference)
import functools

import jax
import jax.numpy as jnp
from jax import lax
from jax.experimental import pallas as pl
from jax.experimental.pallas import tpu as pltpu
from jax.experimental.pallas import tpu_sc as plsc


def _idx_body(tgt_ref, maskT_ref, idsT_ref, gidx_ref, mout_ref, tval_ref,
              *, A, L):
    n = pl.program_id(0)

    @pl.when(n == 0)
    def _():
        ids = idsT_ref[...]
        t0, t1, t2 = tgt_ref[0], tgt_ref[1], tgt_ref[2]
        tm = (ids == t0) | (ids == t1) | (ids == t2)
        aidx = lax.broadcasted_iota(jnp.int32, tm.shape, 0)
        tval_ref[...] = jnp.where(tm, aidx, jnp.int32(A + 1))

    m = maskT_ref[0]
    cand = jnp.where(m, tval_ref[...], jnp.int32(A + 1))
    first = jnp.min(cand, axis=0, keepdims=True)
    valid = first <= jnp.int32(A - 1)
    l = lax.broadcasted_iota(jnp.int32, (1, L), 1)
    base = (n * L + l) * A
    gidx_ref[0] = 3 * (base + jnp.where(valid, first, 0))
    mout_ref[0] = valid.astype(jnp.float32)


def _compute_indices(mask_atoms, atom_name_ids, target_ids):
    N, L, A = mask_atoms.shape
    maskT = jnp.transpose(mask_atoms, (0, 2, 1))
    idsT = atom_name_ids.T
    gidx, mout = pl.pallas_call(
        functools.partial(_idx_body, A=A, L=L),
        grid=(N,),
        in_specs=[
            pl.BlockSpec(memory_space=pltpu.SMEM),
            pl.BlockSpec((1, A, L), lambda n: (n, 0, 0)),
            pl.BlockSpec((A, L), lambda n: (0, 0)),
        ],
        out_specs=[
            pl.BlockSpec((1, 1, L), lambda n: (n, 0, 0)),
            pl.BlockSpec((1, 1, L), lambda n: (n, 0, 0)),
        ],
        out_shape=[
            jax.ShapeDtypeStruct((N, 1, L), jnp.int32),
            jax.ShapeDtypeStruct((N, 1, L), jnp.float32),
        ],
        scratch_shapes=[pltpu.VMEM((A, L), jnp.int32)],
    )(target_ids, maskT, idsT)
    return gidx.reshape(N * L), mout.reshape(N, L)


_CH = 128


def _sc_gather3(pos_1d, idx_flat):
    NL = idx_flat.shape[0]
    mesh = plsc.VectorSubcoreMesh(core_axis_name="c", subcore_axis_name="s")
    nw = mesh.num_cores * mesh.num_subcores
    b = NL // nw
    nch = b // _CH

    @functools.partial(
        pl.kernel,
        mesh=mesh,
        out_type=jax.ShapeDtypeStruct((3 * NL,), jnp.float32),
        scratch_types=[
            pltpu.VMEM((b,), jnp.int32),
            pltpu.VMEM((b,), jnp.int32),
            pltpu.VMEM((b,), jnp.int32),
            pltpu.VMEM((b,), jnp.float32),
            pltpu.VMEM((b,), jnp.float32),
            pltpu.VMEM((b,), jnp.float32),
            pltpu.SemaphoreType.DMA,
        ],
        compiler_params=pltpu.CompilerParams(use_tc_tiling_on_sc=False),
    )
    def k(pos_hbm, idx_hbm, out_hbm, i0, i1, i2, v0, v1, v2, sem):
        wid = lax.axis_index("s") * mesh.num_cores + lax.axis_index("c")
        base = wid * b
        pltpu.sync_copy(idx_hbm.at[pl.ds(base, b)], i0)

        @pl.loop(0, b, step=16)
        def _(i):
            g = i0[pl.ds(i, 16)]
            i1[pl.ds(i, 16)] = g + 1
            i2[pl.ds(i, 16)] = g + 2

        pairs = ((i0, v0), (i1, v1), (i2, v2))
        for iv, vv in pairs:
            @pl.loop(0, nch)
            def _issue(ci, iv=iv, vv=vv):
                o = ci * _CH
                pltpu.async_copy(pos_hbm.at[iv.at[pl.ds(o, _CH)]],
                                 vv.at[pl.ds(o, _CH)], sem)

        for iv, vv in pairs:
            @pl.loop(0, nch)
            def _drain(ci, iv=iv, vv=vv):
                o = ci * _CH
                pltpu.make_async_copy(pos_hbm.at[iv.at[pl.ds(o, _CH)]],
                                      vv.at[pl.ds(o, _CH)], sem).wait()

        pltpu.sync_copy(v0, out_hbm.at[pl.ds(base, b)])
        pltpu.sync_copy(v1, out_hbm.at[pl.ds(NL + base, b)])
        pltpu.sync_copy(v2, out_hbm.at[pl.ds(2 * NL + base, b)])

    return k(pos_1d, idx_flat)


def kernel(pos_atoms, mask_atoms, atom_name_ids, target_ids):
    N, L, A, _ = pos_atoms.shape
    gidx, mout = _compute_indices(mask_atoms, atom_name_ids, target_ids)
    raw = _sc_gather3(pos_atoms.reshape(N * L * A * 3), gidx)
    pos_out = jnp.transpose(raw.reshape(3, N, L), (1, 2, 0)) * mout[:, :, None]
    return pos_out, mout


# speedup vs baseline: 359.5302x; 359.5302x over previous
import functools

import jax
import jax.numpy as jnp
from jax import lax
from jax.experimental import pallas as pl
from jax.experimental.pallas import tpu as pltpu


def _select_body(tgt_ref, planes_ref, maskP_ref, idsP_ref, posP_ref,
                 mout_ref, *, A, NB, LB):
    t0, t1, t2 = tgt_ref[0], tgt_ref[1], tgt_ref[2]
    big = jnp.int32(A + 1)
    first = jnp.full((NB, LB), big, jnp.int32)
    for a in range(A):
        ids_a = idsP_ref[a]
        tm = (ids_a == t0) | (ids_a == t1) | (ids_a == t2)
        sel = maskP_ref[a] & tm[None, :]
        first = jnp.minimum(first, jnp.where(sel, jnp.int32(a), big))
    mout_ref[...] = (first < big).astype(jnp.float32)
    acc0 = jnp.zeros((NB, LB), jnp.float32)
    acc1 = jnp.zeros((NB, LB), jnp.float32)
    acc2 = jnp.zeros((NB, LB), jnp.float32)
    for a in range(A):
        hit = (first == a).astype(jnp.float32)
        acc0 += planes_ref[3 * a] * hit
        acc1 += planes_ref[3 * a + 1] * hit
        acc2 += planes_ref[3 * a + 2] * hit
    posP_ref[0] = acc0
    posP_ref[1] = acc1
    posP_ref[2] = acc2


def kernel(pos_atoms, mask_atoms, atom_name_ids, target_ids):
    N, L, A, _ = pos_atoms.shape
    planes = pos_atoms.transpose(2, 3, 0, 1).reshape(3 * A, N, L)
    maskP = mask_atoms.transpose(2, 0, 1)
    idsP = atom_name_ids.T
    NB, LB = 8, 512
    grid = (L // LB, N // NB)

    posP, mout = pl.pallas_call(
        functools.partial(_select_body, A=A, NB=NB, LB=LB),
        grid=grid,
        in_specs=[
            pl.BlockSpec(memory_space=pltpu.SMEM),
            pl.BlockSpec((3 * A, NB, LB), lambda jl, n: (0, n, jl)),
            pl.BlockSpec((A, NB, LB), lambda jl, n: (0, n, jl)),
            pl.BlockSpec((A, LB), lambda jl, n: (0, jl)),
        ],
        out_specs=[
            pl.BlockSpec((3, NB, LB), lambda jl, n: (0, n, jl)),
            pl.BlockSpec((NB, LB), lambda jl, n: (n, jl)),
        ],
        out_shape=[
            jax.ShapeDtypeStruct((3, N, L), jnp.float32),
            jax.ShapeDtypeStruct((N, L), jnp.float32),
        ],
        compiler_params=pltpu.CompilerParams(
            dimension_semantics=("parallel", "parallel"),
        ),
    )(target_ids, planes, maskP, idsP)

    return posP.transpose(1, 2, 0), mout


# speedup vs baseline: 482.8079x; 1.3429x over previous
import functools

import jax
import jax.numpy as jnp
from jax import lax
from jax.experimental import pallas as pl
from jax.experimental.pallas import tpu as pltpu


def _select_body(tgt_ref, planes_ref, maskP_ref, idsP_ref, posP_ref,
                 mout_ref, *, A, NB, LB):
    t0, t1, t2 = tgt_ref[0], tgt_ref[1], tgt_ref[2]
    big = jnp.int32(A + 1)
    first = jnp.full((NB, LB), big, jnp.int32)
    for a in range(A):
        ids_a = idsP_ref[a]
        tm = (ids_a == t0) | (ids_a == t1) | (ids_a == t2)
        sel = maskP_ref[a] & tm[None, :]
        first = jnp.minimum(first, jnp.where(sel, jnp.int32(a), big))
    mout_ref[...] = (first < big).astype(jnp.float32)
    zero = jnp.zeros((NB, LB), jnp.float32)
    acc0, acc1, acc2 = zero, zero, zero
    for a in range(A):
        hit = first == a
        acc0 = jnp.where(hit, planes_ref[3 * a], acc0)
        acc1 = jnp.where(hit, planes_ref[3 * a + 1], acc1)
        acc2 = jnp.where(hit, planes_ref[3 * a + 2], acc2)
    posP_ref[0] = acc0
    posP_ref[1] = acc1
    posP_ref[2] = acc2


def kernel(pos_atoms, mask_atoms, atom_name_ids, target_ids):
    N, L, A, _ = pos_atoms.shape
    planes = pos_atoms.transpose(2, 3, 0, 1).reshape(3 * A, N, L)
    maskP = mask_atoms.transpose(2, 0, 1)
    idsP = atom_name_ids.T
    NB, LB = 8, 2048
    grid = (L // LB, N // NB)

    posP, mout = pl.pallas_call(
        functools.partial(_select_body, A=A, NB=NB, LB=LB),
        grid=grid,
        in_specs=[
            pl.BlockSpec(memory_space=pltpu.SMEM),
            pl.BlockSpec((3 * A, NB, LB), lambda jl, n: (0, n, jl)),
            pl.BlockSpec((A, NB, LB), lambda jl, n: (0, n, jl)),
            pl.BlockSpec((A, LB), lambda jl, n: (0, jl)),
        ],
        out_specs=[
            pl.BlockSpec((3, NB, LB), lambda jl, n: (0, n, jl)),
            pl.BlockSpec((NB, LB), lambda jl, n: (n, jl)),
        ],
        out_shape=[
            jax.ShapeDtypeStruct((3, N, L), jnp.float32),
            jax.ShapeDtypeStruct((N, L), jnp.float32),
        ],
        compiler_params=pltpu.CompilerParams(
            dimension_semantics=("parallel", "parallel"),
        ),
    )(target_ids, planes, maskP, idsP)

    return posP.transpose(1, 2, 0), mout
